# baseline (speedup 1.0000x reference)
import functools

import jax
import jax.numpy as jnp
from jax import lax
from jax.experimental import pallas as pl
from jax.experimental.pallas import tpu as pltpu
from jax.experimental.pallas import tpu_sc as plsc

B, S, V, D = 4, 8192, 100000, 128
N_TOKENS = B * S
K_QUOTA = max(1, int(0.25 * N_TOKENS))

_NC, _NS = 2, 16
_NW = _NC * _NS
TOK = N_TOKENS // 4
_N_PER_W = TOK // _NW
_CHUNK = 128
_N_CHUNKS = _N_PER_W // _CHUNK


def _rowgather_body(tbl_hbm, idx_hbm, out_hbm, idx_v, buf, gsem, wsem):
    wid = lax.axis_index("s") * _NC + lax.axis_index("c")
    base = wid * _N_PER_W
    pltpu.sync_copy(idx_hbm.at[pl.ds(base, _N_PER_W)], idx_v)

    def gather(j):
        return pltpu.make_async_copy(
            tbl_hbm.at[idx_v.at[pl.ds(j * _CHUNK, _CHUNK)]],
            buf.at[j % 2], gsem)

    def writeback(j):
        return pltpu.make_async_copy(
            buf.at[j % 2], out_hbm.at[pl.ds(base + j * _CHUNK, _CHUNK)], wsem)

    g = {}
    wb = {}
    g[0] = gather(0)
    g[0].start()
    for j in range(_N_CHUNKS):
        if j + 1 < _N_CHUNKS:
            if j >= 1:
                wb[j - 1].wait()
            g[j + 1] = gather(j + 1)
            g[j + 1].start()
        g[j].wait()
        wb[j] = writeback(j)
        wb[j].start()
    wb[_N_CHUNKS - 2].wait()
    wb[_N_CHUNKS - 1].wait()


def _gather_rows(table, idx_flat):
    mesh = plsc.VectorSubcoreMesh(core_axis_name="c", subcore_axis_name="s")
    kern = functools.partial(
        pl.kernel,
        mesh=mesh,
        out_type=jax.ShapeDtypeStruct((TOK, D), jnp.float32),
        scratch_types=[
            pltpu.VMEM((_N_PER_W,), jnp.int32),
            pltpu.VMEM((2, _CHUNK, D), jnp.float32),
            pltpu.SemaphoreType.DMA,
            pltpu.SemaphoreType.DMA,
        ],
    )(_rowgather_body)
    return kern(table, idx_flat)


RB = 8192


def _logits_body(rows_ref, w_ref, out_ref):
    out_ref[...] = lax.dot_general(
        w_ref[...], rows_ref[...], (((1,), (1,)), ((), ())),
        preferred_element_type=jnp.float32)


def _row_logits(rows, w):
    return pl.pallas_call(
        _logits_body,
        grid=(TOK // RB,),
        in_specs=[
            pl.BlockSpec((RB, D), lambda i: (i, 0)),
            pl.BlockSpec((1, D), lambda i: (0, 0)),
        ],
        out_specs=pl.BlockSpec((1, RB), lambda i: (0, i)),
        out_shape=jax.ShapeDtypeStruct((1, TOK), jnp.float32),
    )(rows, w.reshape(1, D))


def _select_body(lg_ref, b_ref, d_ref, mask_ref):
    d = jax.nn.sigmoid(lg_ref[...] + b_ref[0])
    d_ref[...] = d
    keys = lax.bitcast_convert_type(d, jnp.int32)
    t = jnp.int32(0)
    for bit in range(30, -1, -1):
        cand = t | jnp.int32(1 << bit)
        cnt = jnp.sum((keys >= cand).astype(jnp.int32))
        t = jnp.where(cnt >= K_QUOTA, cand, t)
    mask_ref[...] = keys >= t


def _sigmoid_quota_mask(logits, b):
    return pl.pallas_call(
        _select_body,
        in_specs=[
            pl.BlockSpec((B, S), lambda: (0, 0)),
            pl.BlockSpec(memory_space=pltpu.SMEM),
        ],
        out_specs=[
            pl.BlockSpec((B, S), lambda: (0, 0)),
            pl.BlockSpec((B, S), lambda: (0, 0)),
        ],
        out_shape=[
            jax.ShapeDtypeStruct((B, S), jnp.float32),
            jax.ShapeDtypeStruct((B, S), jnp.bool_),
        ],
    )(logits, b.reshape(1))


N_SPLIT = 4
TOK_CHUNK = N_TOKENS // N_SPLIT


def kernel(input_ids, table, w, b):
    idx = input_ids.reshape(-1)
    lg = []
    for c in range(N_SPLIT):
        rows = _gather_rows(table, lax.dynamic_slice(idx, (c * TOK_CHUNK,), (TOK_CHUNK,)))
        lg.append(_row_logits(rows, w))
    logits = jnp.concatenate(lg, axis=1).reshape(B, S)
    difficulty, mask = _sigmoid_quota_mask(logits, b)
    info_k = jnp.array(K_QUOTA, dtype=jnp.int32)
    return difficulty, mask, info_k


# speedup vs baseline: 1.2513x; 1.2513x over previous
import functools

import jax
import jax.numpy as jnp
from jax import lax
from jax.experimental import pallas as pl
from jax.experimental.pallas import tpu as pltpu
from jax.experimental.pallas import tpu_sc as plsc

B, S, V, D = 4, 8192, 100000, 128
N_TOKENS = B * S
K_QUOTA = max(1, int(0.25 * N_TOKENS))


VB = 25088
V_PAD = ((V + VB - 1) // VB) * VB


def _scores_body(tbl_ref, w_ref, out_ref):
    out_ref[...] = lax.dot_general(
        w_ref[...], tbl_ref[...], (((1,), (1,)), ((), ())),
        preferred_element_type=jnp.float32)


def _vocab_scores(table, w):
    grid = V_PAD // VB
    return pl.pallas_call(
        _scores_body,
        grid=(grid,),
        in_specs=[
            pl.BlockSpec((VB, D), lambda i: (i, 0)),
            pl.BlockSpec((1, D), lambda i: (0, 0)),
        ],
        out_specs=pl.BlockSpec((1, VB), lambda i: (0, i)),
        out_shape=jax.ShapeDtypeStruct((1, V_PAD), jnp.float32),
    )(table, w.reshape(1, D)).reshape(-1)


_NC, _NS = 2, 16
_NW = _NC * _NS
_N_PER_W = N_TOKENS // _NW
_CHUNK = 128
_N_CHUNKS = _N_PER_W // _CHUNK


def _gather_body(scores_hbm, idx_hbm, out_hbm, idx_v, val_v, sem):
    wid = lax.axis_index("s") * _NC + lax.axis_index("c")
    base = wid * _N_PER_W
    pltpu.sync_copy(idx_hbm.at[pl.ds(base, _N_PER_W)], idx_v)
    copies = []
    for j in range(_N_CHUNKS):
        c = pltpu.make_async_copy(
            scores_hbm.at[idx_v.at[pl.ds(j * _CHUNK, _CHUNK)]],
            val_v.at[pl.ds(j * _CHUNK, _CHUNK)],
            sem,
        )
        c.start()
        copies.append(c)
    for c in copies:
        c.wait()
    pltpu.sync_copy(val_v, out_hbm.at[pl.ds(base, _N_PER_W)])


def _gather_scores(scores, idx_flat):
    mesh = plsc.VectorSubcoreMesh(core_axis_name="c", subcore_axis_name="s")
    kern = functools.partial(
        pl.kernel,
        mesh=mesh,
        out_type=jax.ShapeDtypeStruct((N_TOKENS,), jnp.float32),
        scratch_types=[
            pltpu.VMEM((_N_PER_W,), jnp.int32),
            pltpu.VMEM((_N_PER_W,), jnp.float32),
            pltpu.SemaphoreType.DMA,
        ],
    )(_gather_body)
    return kern(scores, idx_flat)


def _select_body(lg_ref, b_ref, d_ref, mask_ref):
    d = jax.nn.sigmoid(lg_ref[...] + b_ref[0])
    d_ref[...] = d
    keys = lax.bitcast_convert_type(d, jnp.int32)
    t = jnp.int32(0)
    for bit in range(29, -1, -1):
        cand = t | jnp.int32(1 << bit)
        cnt = jnp.sum((keys >= cand).astype(jnp.int32))
        t = jnp.where(cnt >= K_QUOTA, cand, t)
    mask_ref[...] = keys >= t


def _sigmoid_quota_mask(logits, b):
    return pl.pallas_call(
        _select_body,
        in_specs=[
            pl.BlockSpec((B, S), lambda: (0, 0)),
            pl.BlockSpec(memory_space=pltpu.SMEM),
        ],
        out_specs=[
            pl.BlockSpec((B, S), lambda: (0, 0)),
            pl.BlockSpec((B, S), lambda: (0, 0)),
        ],
        out_shape=[
            jax.ShapeDtypeStruct((B, S), jnp.float32),
            jax.ShapeDtypeStruct((B, S), jnp.bool_),
        ],
    )(logits, b.reshape(1))


def kernel(input_ids, table, w, b):
    scores = _vocab_scores(table, w)
    logits = _gather_scores(scores, input_ids.reshape(-1)).reshape(B, S)
    difficulty, mask = _sigmoid_quota_mask(logits, b)
    info_k = jnp.array(K_QUOTA, dtype=jnp.int32)
    return difficulty, mask, info_k


# speedup vs baseline: 1.2695x; 1.0145x over previous
import functools

import jax
import jax.numpy as jnp
from jax import lax
from jax.experimental import pallas as pl
from jax.experimental.pallas import tpu as pltpu
from jax.experimental.pallas import tpu_sc as plsc

B, S, V, D = 4, 8192, 100000, 128
N_TOKENS = B * S
K_QUOTA = max(1, int(0.25 * N_TOKENS))


VB = 25088
V_PAD = ((V + VB - 1) // VB) * VB


def _scores_body(tbl_ref, w_ref, out_ref):
    out_ref[...] = lax.dot_general(
        w_ref[...], tbl_ref[...], (((1,), (1,)), ((), ())),
        preferred_element_type=jnp.float32)


def _vocab_scores(table, w):
    grid = V_PAD // VB
    return pl.pallas_call(
        _scores_body,
        grid=(grid,),
        in_specs=[
            pl.BlockSpec((VB, D), lambda i: (i, 0)),
            pl.BlockSpec((1, D), lambda i: (0, 0)),
        ],
        out_specs=pl.BlockSpec((1, VB), lambda i: (0, i)),
        out_shape=jax.ShapeDtypeStruct((1, V_PAD), jnp.float32),
    )(table, w.reshape(1, D)).reshape(-1)


_NC, _NS = 2, 16
_NW = _NC * _NS
_N_PER_W = N_TOKENS // _NW
_CHUNK = 128
_N_CHUNKS = _N_PER_W // _CHUNK


def _gather_body(scores_hbm, idx_hbm, out_hbm, idx_v, val_v, sem):
    wid = lax.axis_index("s") * _NC + lax.axis_index("c")
    base = wid * _N_PER_W
    pltpu.sync_copy(idx_hbm.at[pl.ds(base, _N_PER_W)], idx_v)
    copies = []
    for j in range(_N_CHUNKS):
        c = pltpu.make_async_copy(
            scores_hbm.at[idx_v.at[pl.ds(j * _CHUNK, _CHUNK)]],
            val_v.at[pl.ds(j * _CHUNK, _CHUNK)],
            sem,
        )
        c.start()
        copies.append(c)
    for c in copies:
        c.wait()
    pltpu.sync_copy(val_v, out_hbm.at[pl.ds(base, _N_PER_W)])


def _gather_scores(scores, idx_flat):
    mesh = plsc.VectorSubcoreMesh(core_axis_name="c", subcore_axis_name="s")
    kern = functools.partial(
        pl.kernel,
        mesh=mesh,
        out_type=jax.ShapeDtypeStruct((N_TOKENS,), jnp.float32),
        scratch_types=[
            pltpu.VMEM((_N_PER_W,), jnp.int32),
            pltpu.VMEM((_N_PER_W,), jnp.float32),
            pltpu.SemaphoreType.DMA,
        ],
    )(_gather_body)
    return kern(scores, idx_flat)


def _select_body(lg_ref, b_ref, d_ref, mask_ref):
    d = jax.nn.sigmoid(lg_ref[...] + b_ref[0])
    d_ref[...] = d
    keys = lax.bitcast_convert_type(d, jnp.int32)
    t = jnp.int32(0)
    for shift in range(28, -1, -2):
        digit = jnp.int32(0)
        for m in range(1, 4):
            cand = t | jnp.int32(m << shift)
            cnt = jnp.sum((keys >= cand).astype(jnp.int32))
            digit = digit + (cnt >= K_QUOTA).astype(jnp.int32)
        t = t | (digit << shift)
    mask_ref[...] = keys >= t


def _sigmoid_quota_mask(logits, b):
    return pl.pallas_call(
        _select_body,
        in_specs=[
            pl.BlockSpec((B, S), lambda: (0, 0)),
            pl.BlockSpec(memory_space=pltpu.SMEM),
        ],
        out_specs=[
            pl.BlockSpec((B, S), lambda: (0, 0)),
            pl.BlockSpec((B, S), lambda: (0, 0)),
        ],
        out_shape=[
            jax.ShapeDtypeStruct((B, S), jnp.float32),
            jax.ShapeDtypeStruct((B, S), jnp.bool_),
        ],
    )(logits, b.reshape(1))


def kernel(input_ids, table, w, b):
    scores = _vocab_scores(table, w)
    logits = _gather_scores(scores, input_ids.reshape(-1)).reshape(B, S)
    difficulty, mask = _sigmoid_quota_mask(logits, b)
    info_k = jnp.array(K_QUOTA, dtype=jnp.int32)
    return difficulty, mask, info_k
